# Initial kernel scaffold; baseline (speedup 1.0000x reference)
import jax
import jax.numpy as jnp
from jax.experimental import pallas as pl


def kernel(adj, node_emb, W1, b1, W2, b2, fc1_W, fc1_b, fc2_W, fc2_b):
    raise NotImplementedError("write your pallas kernel here")


# speedup vs baseline: 1.2264x; 1.2264x over previous
import jax
import jax.numpy as jnp
from jax.experimental import pallas as pl
from jax.experimental.pallas import tpu as pltpu

N = 1024


def _fused_kernel(adj_ref, emb_ref, w1_ref, b1_ref, w2_ref, b2_ref,
                  fc1w_ref, fc1b_ref, fc2w_ref, fc2b_ref, x_out_ref, a2_out_ref):
    adj = adj_ref[...]
    row = jax.lax.broadcasted_iota(jnp.int32, (N, N), 0)
    col = jax.lax.broadcasted_iota(jnp.int32, (N, N), 1)
    a2 = adj + jnp.where(row == col, 2.0, 0.0).astype(jnp.float32)

    ones_col = jnp.ones((N, 1), jnp.float32)
    tdims = (((0,), (0,)), ((), ()))
    deg = jax.lax.dot_general(a2, ones_col, tdims,
                              preferred_element_type=jnp.float32)
    dis = jnp.where(deg > 0, jax.lax.rsqrt(deg), 0.0)

    xt = jnp.dot(emb_ref[...], w1_ref[...], preferred_element_type=jnp.float32)
    z = jax.lax.dot_general(a2, dis * xt, tdims,
                            preferred_element_type=jnp.float32)
    x = jnp.maximum(dis * z + b1_ref[...], 0.0)

    xt2 = jnp.dot(x, w2_ref[...], preferred_element_type=jnp.float32)
    z2 = jax.lax.dot_general(a2, dis * xt2, tdims,
                             preferred_element_type=jnp.float32)
    x2 = jnp.maximum(dis * z2 + b2_ref[...], 0.0)
    x_out_ref[...] = x2

    h = jnp.maximum(jnp.dot(x2, fc1w_ref[...], preferred_element_type=jnp.float32)
                    + fc1b_ref[...], 0.0)
    a2_out_ref[...] = (jnp.dot(h, fc2w_ref[...], preferred_element_type=jnp.float32)
                       + fc2b_ref[...])


def kernel(adj, node_emb, W1, b1, W2, b2, fc1_W, fc1_b, fc2_W, fc2_b):
    x, a2 = pl.pallas_call(
        _fused_kernel,
        out_shape=(
            jax.ShapeDtypeStruct((N, 128), jnp.float32),
            jax.ShapeDtypeStruct((N, 1), jnp.float32),
        ),
    )(adj, node_emb, W1, b1.reshape(1, 512), W2, b2.reshape(1, 128),
      fc1_W, fc1_b.reshape(1, 256), fc2_W, fc2_b.reshape(1, 1))
    return (x, a2)
